# baseline (speedup 1.0000x reference)
import functools

import jax
import jax.numpy as jnp
import numpy as np
from jax import lax
from jax.experimental import pallas as pl
from jax.experimental.pallas import tpu as pltpu
from jax.experimental.pallas import tpu_sc as plsc

B = 4096
L = 20
NTAB = 26
VOCAB = 100000
M = 32
NC = 2
NS = 16
NW = NC * NS
BAGS_W = B // NW
CH = BAGS_W // 2
CH_ROWS = CH * L
NSUB = CH_ROWS // 128


def _sc_embed_body(idx_hbm, emb_hbm, out_hbm,
                   idx_v0, idx_v1, rows0, rows1, out_v, sem0, sem1):
    w = lax.axis_index("s") * NC + lax.axis_index("c")

    def fire(chunk, idx_v, rows, sem):
        pltpu.sync_copy(idx_hbm.at[chunk], idx_v)
        for c in range(NSUB):
            pltpu.async_copy(emb_hbm.at[idx_v.at[c]],
                             rows.at[pl.ds(c * 128, 128)], sem)

    def drain(idx_v, rows, sem):
        for c in range(NSUB):
            pltpu.make_async_copy(emb_hbm.at[idx_v.at[0]],
                                  rows.at[pl.ds(c * 128, 128)], sem).wait()

    hi_mask = jnp.full((16,), -65536, jnp.int32)
    sh16 = jnp.full((16,), 16, jnp.int32)

    def split_row(vi):
        even = lax.bitcast_convert_type(lax.shift_left(vi, sh16), jnp.float32)
        odd = lax.bitcast_convert_type(jnp.bitwise_and(vi, hi_mask), jnp.float32)
        return even, odd

    def pool(rows, out_base):
        def bag_body(b, carry):
            base = b * L
            a0, a1 = split_row(rows[base, :])
            for r in range(1, L):
                b0, b1 = split_row(rows[base + r, :])
                a0 = a0 + b0
                a1 = a1 + b1
            out_v[out_base + b, pl.ds(0, 16)] = a0
            out_v[out_base + b, pl.ds(16, 16)] = a1
            return carry
        lax.fori_loop(0, CH, bag_body, 0, unroll=False)

    fire(w * 2, idx_v0, rows0, sem0)

    def table_body(t, carry):
        tw2 = (t * NW + w) * 2
        fire(tw2 + 1, idx_v1, rows1, sem1)
        drain(idx_v0, rows0, sem0)
        pool(rows0, 0)

        @pl.when(t < NTAB - 1)
        def _():
            fire(tw2 + 2 * NW, idx_v0, rows0, sem0)

        drain(idx_v1, rows1, sem1)
        pool(rows1, CH)
        pltpu.sync_copy(out_v, out_hbm.at[t, pl.ds(w * BAGS_W, BAGS_W), :])
        return carry

    lax.fori_loop(0, NTAB, table_body, 0, unroll=False)


def _sc_embed(idx4, emb_flat):
    mesh = plsc.VectorSubcoreMesh(core_axis_name="c", subcore_axis_name="s")
    kfn = pl.kernel(
        _sc_embed_body,
        out_type=jax.ShapeDtypeStruct((NTAB, B, M), jnp.float32),
        mesh=mesh,
        scratch_types=[
            pltpu.VMEM((NSUB, 128), jnp.int32),
            pltpu.VMEM((NSUB, 128), jnp.int32),
            pltpu.VMEM((CH_ROWS, M // 2), jnp.int32),
            pltpu.VMEM((CH_ROWS, M // 2), jnp.int32),
            pltpu.VMEM((BAGS_W, M), jnp.float32),
            pltpu.SemaphoreType.DMA,
            pltpu.SemaphoreType.DMA,
        ],
        compiler_params=pltpu.CompilerParams(use_tc_tiling_on_sc=False),
    )
    return kfn(idx4, emb_flat)


JB = VOCAB
JQ = JB // 4


def _tc_transpose_body(src_ref, out_ref):
    x = src_ref[0].astype(jnp.bfloat16)
    ch = JQ // 8
    for c in range(8):
        xs = jnp.concatenate(
            [x[:, dj * JQ + c * ch:dj * JQ + (c + 1) * ch] for dj in range(4)],
            axis=0)
        out_ref[c * ch:(c + 1) * ch, :] = xs.T


def _tc_transpose(emb_t):
    return pl.pallas_call(
        _tc_transpose_body,
        grid=(NTAB,),
        in_specs=[pl.BlockSpec((1, M, JB), lambda t: (t, 0, 0))],
        out_specs=pl.BlockSpec((JQ, 4 * M), lambda t: (t, 0)),
        out_shape=jax.ShapeDtypeStruct((NTAB * VOCAB // 4, 4 * M), jnp.bfloat16),
        compiler_params=pltpu.CompilerParams(vmem_limit_bytes=100 * 1024 * 1024),
    )(emb_t)


def _permute_idx(idx):
    b, l = idx // JB, idx % JB
    return (b * JQ + l % JQ) * 4 + l // JQ


_SIGMA = np.concatenate([np.arange(0, M, 2), np.arange(1, M, 2)])


def _tc_dense_body(dense_t, ly_bm,
                   w0, b0, w1, b1, w2, b2,
                   tw0, tb0, tw1, tb1, tw2, tb2, out_ref):
    d = dense_t[...]
    x = jax.nn.relu(jnp.dot(w0[...], d, preferred_element_type=jnp.float32)
                    + b0[...][:, None])
    x = jax.nn.relu(jnp.dot(w1[...], x, preferred_element_type=jnp.float32)
                    + b1[...][:, None])
    x = jax.nn.relu(jnp.dot(w2[...], x, preferred_element_type=jnp.float32)
                    + b2[...][:, None])
    ly3 = ly_bm[...]
    ly_t = jnp.transpose(ly3, (0, 2, 1)).reshape(NTAB * M, ly3.shape[1])
    r = jnp.concatenate([x, ly_t], axis=0)
    blk = r.shape[1]
    pieces = [x]
    for i in range(1, NTAB + 1):
        u = r[i * M:(i + 1) * M]
        r3 = r[:i * M].reshape(i, M, blk)
        pi = (r3 * u[None]).sum(axis=1)
        pieces.append(pi)
    z = jnp.concatenate(pieces, axis=0)
    h = jax.nn.relu(jnp.dot(tw0[...], z, preferred_element_type=jnp.float32)
                    + tb0[...][:, None])
    h = jax.nn.relu(jnp.dot(tw1[...], h, preferred_element_type=jnp.float32)
                    + tb1[...][:, None])
    p = jax.nn.sigmoid(jnp.dot(tw2[...], h, preferred_element_type=jnp.float32)
                       + tb2[...][:, None])
    out_ref[...] = p.T


def _tc_dense(dense_t, ly_bm, weights):
    blk = 512
    grid = (B // blk,)
    full = lambda shape: pl.BlockSpec(shape, lambda i: (0,) * len(shape))
    in_specs = [
        pl.BlockSpec((13, blk), lambda i: (0, i)),
        pl.BlockSpec((NTAB, blk, M), lambda i: (0, i, 0)),
    ] + [full(w.shape) for w in weights]
    out_specs = pl.BlockSpec((blk, 1), lambda i: (i, 0))
    return pl.pallas_call(
        _tc_dense_body,
        grid=grid,
        in_specs=in_specs,
        out_specs=out_specs,
        out_shape=jax.ShapeDtypeStruct((B, 1), jnp.float32),
    )(dense_t, ly_bm, *weights)


def kernel(dense_x, lS_i, lS_o, emb_tables,
           bot_W0, bot_b0, bot_W1, bot_b1, bot_W2, bot_b2,
           top_W0, top_b0, top_W1, top_b1, top_W2, top_b2):
    del lS_o
    offs = (jnp.arange(NTAB, dtype=jnp.int32) * VOCAB)[:, None]
    idx4 = _permute_idx(lS_i) + offs
    idx4 = idx4.reshape(NTAB, NW, 2, NSUB, 128).reshape(
        NTAB * NW * 2, NSUB, 128)
    emb_t = jnp.transpose(emb_tables, (0, 2, 1))
    emb_bf = _tc_transpose(emb_t).reshape(NTAB * VOCAB, M // 2, 2)
    emb_flat = lax.bitcast_convert_type(emb_bf, jnp.int32)
    ly_bm = _sc_embed(idx4, emb_flat)
    dense_t = dense_x.T
    sig = jnp.asarray(_SIGMA)
    weights = (bot_W0, bot_b0, bot_W1, bot_b1, bot_W2[sig], bot_b2[sig],
               jnp.concatenate([top_W0[:, :M][:, sig], top_W0[:, M:]], axis=1),
               top_b0, top_W1, top_b1, top_W2, top_b2)
    return _tc_dense(dense_t, ly_bm, weights)


# speedup vs baseline: 121.8958x; 121.8958x over previous
import functools

import jax
import jax.numpy as jnp
import numpy as np
from jax import lax
from jax.experimental import pallas as pl
from jax.experimental.pallas import tpu as pltpu
from jax.experimental.pallas import tpu_sc as plsc

B = 4096
L = 20
NTAB = 26
VOCAB = 100000
M = 32
NC = 2
NS = 16
NW = NC * NS
BAGS_W = B // NW
CH = BAGS_W // 2
CH_ROWS = CH * L
NSUB = CH_ROWS // 128


def _sc_embed_body(idx_hbm, emb_hbm, out_hbm,
                   idx_v0, idx_v1, rows0, rows1, out_v, sem0, sem1):
    w = lax.axis_index("s") * NC + lax.axis_index("c")

    def fire(chunk, idx_v, rows, sem):
        pltpu.sync_copy(idx_hbm.at[chunk], idx_v)
        for c in range(NSUB):
            pltpu.async_copy(emb_hbm.at[idx_v.at[c]],
                             rows.at[pl.ds(c * 128, 128)], sem)

    def drain(idx_v, rows, sem):
        for c in range(NSUB):
            pltpu.make_async_copy(emb_hbm.at[idx_v.at[0]],
                                  rows.at[pl.ds(c * 128, 128)], sem).wait()

    hi_mask = jnp.full((16,), -65536, jnp.int32)
    sh16 = jnp.full((16,), 16, jnp.int32)

    def split_row(vi):
        even = lax.bitcast_convert_type(lax.shift_left(vi, sh16), jnp.float32)
        odd = lax.bitcast_convert_type(jnp.bitwise_and(vi, hi_mask), jnp.float32)
        return even, odd

    def pool(rows, out_base):
        def bag_body(b, carry):
            base = b * L
            a0, a1 = split_row(rows[base, :])
            for r in range(1, L):
                b0, b1 = split_row(rows[base + r, :])
                a0 = a0 + b0
                a1 = a1 + b1
            out_v[out_base + b, pl.ds(0, 16)] = a0
            out_v[out_base + b, pl.ds(16, 16)] = a1
            return carry
        lax.fori_loop(0, CH, bag_body, 0, unroll=False)

    fire(w * 2, idx_v0, rows0, sem0)

    def table_body(t, carry):
        tw2 = (t * NW + w) * 2
        fire(tw2 + 1, idx_v1, rows1, sem1)
        drain(idx_v0, rows0, sem0)
        pool(rows0, 0)

        @pl.when(t < NTAB - 1)
        def _():
            fire(tw2 + 2 * NW, idx_v0, rows0, sem0)

        drain(idx_v1, rows1, sem1)
        pool(rows1, CH)
        pltpu.sync_copy(out_v, out_hbm.at[t, pl.ds(w * BAGS_W, BAGS_W), :])
        return carry

    lax.fori_loop(0, NTAB, table_body, 0, unroll=False)


def _sc_embed(idx4, emb_flat):
    mesh = plsc.VectorSubcoreMesh(core_axis_name="c", subcore_axis_name="s")
    kfn = pl.kernel(
        _sc_embed_body,
        out_type=jax.ShapeDtypeStruct((NTAB, B, M), jnp.float32),
        mesh=mesh,
        scratch_types=[
            pltpu.VMEM((NSUB, 128), jnp.int32),
            pltpu.VMEM((NSUB, 128), jnp.int32),
            pltpu.VMEM((CH_ROWS, M // 2), jnp.int32),
            pltpu.VMEM((CH_ROWS, M // 2), jnp.int32),
            pltpu.VMEM((BAGS_W, M), jnp.float32),
            pltpu.SemaphoreType.DMA,
            pltpu.SemaphoreType.DMA,
        ],
        compiler_params=pltpu.CompilerParams(use_tc_tiling_on_sc=False),
    )
    return kfn(idx4, emb_flat)


JB = VOCAB
D8 = 8
SEG = JB // D8


def _tc_transpose_body(src_ref, out_ref):
    x = src_ref[0]
    v = lax.bitcast_convert_type(x, jnp.int32)
    rnd = v + 32768
    lo = lax.shift_right_logical(rnd[:M // 2], 16)
    hi = jnp.bitwise_and(rnd[M // 2:], -65536)
    w = jnp.bitwise_or(lo, hi)
    ch = SEG // 4
    for c in range(4):
        xs = jnp.concatenate(
            [w[:, dq * SEG + c * ch:dq * SEG + (c + 1) * ch] for dq in range(D8)],
            axis=0)
        out_ref[0, c * ch:(c + 1) * ch, :] = xs.T


def _tc_transpose(emb_t):
    return pl.pallas_call(
        _tc_transpose_body,
        grid=(NTAB,),
        in_specs=[pl.BlockSpec((1, M, JB), lambda t: (t, 0, 0))],
        out_specs=pl.BlockSpec((1, SEG, 128), lambda t: (t, 0, 0)),
        out_shape=jax.ShapeDtypeStruct((NTAB, SEG, 128), jnp.int32),
        compiler_params=pltpu.CompilerParams(vmem_limit_bytes=100 * 1024 * 1024),
    )(emb_t)


def _permute_idx(idx):
    b, l = idx // JB, idx % JB
    return (b * SEG + l % SEG) * D8 + l // SEG


def _tc_dense_body(dense_t, ly_bm,
                   w0, b0, w1, b1, w2, b2,
                   tw0, tb0, tw1, tb1, tw2, tb2, out_ref):
    d = dense_t[...]
    x = jax.nn.relu(jnp.dot(w0[...], d, preferred_element_type=jnp.float32)
                    + b0[...][:, None])
    x = jax.nn.relu(jnp.dot(w1[...], x, preferred_element_type=jnp.float32)
                    + b1[...][:, None])
    x = jax.nn.relu(jnp.dot(w2[...], x, preferred_element_type=jnp.float32)
                    + b2[...][:, None])
    ly3 = ly_bm[...]
    ly_t = jnp.transpose(ly3, (0, 2, 1)).reshape(NTAB * M, ly3.shape[1])
    r = jnp.concatenate([x, ly_t], axis=0)
    blk = r.shape[1]
    pieces = [x]
    for i in range(1, NTAB + 1):
        u = r[i * M:(i + 1) * M]
        r3 = r[:i * M].reshape(i, M, blk)
        pi = (r3 * u[None]).sum(axis=1)
        pieces.append(pi)
    z = jnp.concatenate(pieces, axis=0)
    h = jax.nn.relu(jnp.dot(tw0[...], z, preferred_element_type=jnp.float32)
                    + tb0[...][:, None])
    h = jax.nn.relu(jnp.dot(tw1[...], h, preferred_element_type=jnp.float32)
                    + tb1[...][:, None])
    p = jax.nn.sigmoid(jnp.dot(tw2[...], h, preferred_element_type=jnp.float32)
                       + tb2[...][:, None])
    out_ref[...] = p.T


def _tc_dense(dense_t, ly_bm, weights):
    blk = 512
    grid = (B // blk,)
    full = lambda shape: pl.BlockSpec(shape, lambda i: (0,) * len(shape))
    in_specs = [
        pl.BlockSpec((13, blk), lambda i: (0, i)),
        pl.BlockSpec((NTAB, blk, M), lambda i: (0, i, 0)),
    ] + [full(w.shape) for w in weights]
    out_specs = pl.BlockSpec((blk, 1), lambda i: (i, 0))
    return pl.pallas_call(
        _tc_dense_body,
        grid=grid,
        in_specs=in_specs,
        out_specs=out_specs,
        out_shape=jax.ShapeDtypeStruct((B, 1), jnp.float32),
    )(dense_t, ly_bm, *weights)


def kernel(dense_x, lS_i, lS_o, emb_tables,
           bot_W0, bot_b0, bot_W1, bot_b1, bot_W2, bot_b2,
           top_W0, top_b0, top_W1, top_b1, top_W2, top_b2):
    del lS_o
    offs = (jnp.arange(NTAB, dtype=jnp.int32) * VOCAB)[:, None]
    idx4 = _permute_idx(lS_i) + offs
    idx4 = idx4.reshape(NTAB, NW, 2, NSUB, 128).reshape(
        NTAB * NW * 2, NSUB, 128)
    emb_t = jnp.transpose(emb_tables, (0, 2, 1))
    emb_flat = _tc_transpose(emb_t).reshape(NTAB * VOCAB, M // 2)
    ly_bm = _sc_embed(idx4, emb_flat)
    dense_t = dense_x.T
    weights = (bot_W0, bot_b0, bot_W1, bot_b1, bot_W2, bot_b2,
               top_W0, top_b0, top_W1, top_b1, top_W2, top_b2)
    return _tc_dense(dense_t, ly_bm, weights)


# speedup vs baseline: 163.2185x; 1.3390x over previous
import functools

import jax
import jax.numpy as jnp
import numpy as np
from jax import lax
from jax.experimental import pallas as pl
from jax.experimental.pallas import tpu as pltpu
from jax.experimental.pallas import tpu_sc as plsc

B = 4096
L = 20
NTAB = 26
VOCAB = 100000
M = 32
NC = 2
NS = 16
NW = NC * NS
BAGS_W = B // NW
CH = BAGS_W // 2
CH_ROWS = CH * L
NSUB = CH_ROWS // 128


def _sc_embed_body(idx_hbm, emb_hbm, out_hbm,
                   idx_v0, idx_v1, rows0, rows1, out_v, sem0, sem1):
    w = lax.axis_index("s") * NC + lax.axis_index("c")

    def fire(chunk, idx_v, rows, sem):
        pltpu.sync_copy(idx_hbm.at[chunk], idx_v)
        for c in range(NSUB):
            pltpu.async_copy(emb_hbm.at[idx_v.at[c]],
                             rows.at[pl.ds(c * 128, 128)], sem)

    def drain(idx_v, rows, sem):
        for c in range(NSUB):
            pltpu.make_async_copy(emb_hbm.at[idx_v.at[0]],
                                  rows.at[pl.ds(c * 128, 128)], sem).wait()

    hi_mask = jnp.full((16,), -65536, jnp.int32)
    sh16 = jnp.full((16,), 16, jnp.int32)

    def split_row(vi):
        even = lax.bitcast_convert_type(lax.shift_left(vi, sh16), jnp.float32)
        odd = lax.bitcast_convert_type(jnp.bitwise_and(vi, hi_mask), jnp.float32)
        return even, odd

    def pool(rows, out_base):
        def bag_body(b, carry):
            base = b * L
            a0, a1 = split_row(rows[base, :])
            for r in range(1, L):
                b0, b1 = split_row(rows[base + r, :])
                a0 = a0 + b0
                a1 = a1 + b1
            out_v[out_base + b, pl.ds(0, 16)] = a0
            out_v[out_base + b, pl.ds(16, 16)] = a1
            return carry
        lax.fori_loop(0, CH, bag_body, 0, unroll=False)

    fire(w * 2, idx_v0, rows0, sem0)

    def table_body(t, carry):
        tw2 = (t * NW + w) * 2
        fire(tw2 + 1, idx_v1, rows1, sem1)
        drain(idx_v0, rows0, sem0)
        pool(rows0, 0)

        @pl.when(t < NTAB - 1)
        def _():
            fire(tw2 + 2 * NW, idx_v0, rows0, sem0)

        drain(idx_v1, rows1, sem1)
        pool(rows1, CH)
        pltpu.sync_copy(out_v, out_hbm.at[t, pl.ds(w * BAGS_W, BAGS_W), :])
        return carry

    lax.fori_loop(0, NTAB, table_body, 0, unroll=False)


def _sc_embed(idx4, emb_flat):
    mesh = plsc.VectorSubcoreMesh(core_axis_name="c", subcore_axis_name="s")
    kfn = pl.kernel(
        _sc_embed_body,
        out_type=jax.ShapeDtypeStruct((NTAB, B, M), jnp.float32),
        mesh=mesh,
        scratch_types=[
            pltpu.VMEM((NSUB, 128), jnp.int32),
            pltpu.VMEM((NSUB, 128), jnp.int32),
            pltpu.VMEM((CH_ROWS, M // 2), jnp.int32),
            pltpu.VMEM((CH_ROWS, M // 2), jnp.int32),
            pltpu.VMEM((BAGS_W, M), jnp.float32),
            pltpu.SemaphoreType.DMA,
            pltpu.SemaphoreType.DMA,
        ],
        compiler_params=pltpu.CompilerParams(use_tc_tiling_on_sc=False),
    )
    return kfn(idx4, emb_flat)


JB = VOCAB
D8 = 8
SEG = JB // D8
SEGP = SEG + 4


def _tc_transpose_body(src_ref, out_ref):
    x = src_ref[0]
    v = lax.bitcast_convert_type(x, jnp.int32)
    rnd = v + 32768
    lo = lax.shift_right_logical(rnd[:M // 2], 16)
    hi = jnp.bitwise_and(rnd[M // 2:], -65536)
    w = jnp.bitwise_or(lo, hi)
    ch = SEG // 4
    for c in range(4):
        xs = jnp.concatenate(
            [w[:, dq * SEG + c * ch:dq * SEG + (c + 1) * ch] for dq in range(D8)],
            axis=0)
        out_ref[0, c * ch:(c + 1) * ch, :] = xs.T


def _tc_transpose(emb_t):
    return pl.pallas_call(
        _tc_transpose_body,
        grid=(NTAB,),
        in_specs=[pl.BlockSpec((1, M, JB), lambda t: (t, 0, 0))],
        out_specs=pl.BlockSpec((1, SEGP, 128), lambda t: (t, 0, 0)),
        out_shape=jax.ShapeDtypeStruct((NTAB, SEGP, 128), jnp.int32),
        compiler_params=pltpu.CompilerParams(vmem_limit_bytes=100 * 1024 * 1024),
    )(emb_t)


def _permute_idx(idx):
    return (idx % SEG) * D8 + idx // SEG


def _tc_dense_body(dense_t, ly_bm,
                   w0, b0, w1, b1, w2, b2,
                   tw0, tb0, tw1, tb1, tw2, tb2, out_ref):
    d = dense_t[...]
    x = jax.nn.relu(jnp.dot(w0[...], d, preferred_element_type=jnp.float32)
                    + b0[...][:, None])
    x = jax.nn.relu(jnp.dot(w1[...], x, preferred_element_type=jnp.float32)
                    + b1[...][:, None])
    x = jax.nn.relu(jnp.dot(w2[...], x, preferred_element_type=jnp.float32)
                    + b2[...][:, None])
    ly3 = ly_bm[...]
    ly_t = jnp.transpose(ly3, (0, 2, 1)).reshape(NTAB * M, ly3.shape[1])
    r = jnp.concatenate([x, ly_t], axis=0)
    blk = r.shape[1]
    pieces = [x]
    for i in range(1, NTAB + 1):
        u = r[i * M:(i + 1) * M]
        r3 = r[:i * M].reshape(i, M, blk)
        pi = (r3 * u[None]).sum(axis=1)
        pieces.append(pi)
    z = jnp.concatenate(pieces, axis=0)
    h = jax.nn.relu(jnp.dot(tw0[...], z, preferred_element_type=jnp.float32)
                    + tb0[...][:, None])
    h = jax.nn.relu(jnp.dot(tw1[...], h, preferred_element_type=jnp.float32)
                    + tb1[...][:, None])
    p = jax.nn.sigmoid(jnp.dot(tw2[...], h, preferred_element_type=jnp.float32)
                       + tb2[...][:, None])
    out_ref[...] = p.T


def _tc_dense(dense_t, ly_bm, weights):
    blk = 512
    grid = (B // blk,)
    full = lambda shape: pl.BlockSpec(shape, lambda i: (0,) * len(shape))
    in_specs = [
        pl.BlockSpec((13, blk), lambda i: (0, i)),
        pl.BlockSpec((NTAB, blk, M), lambda i: (0, i, 0)),
    ] + [full(w.shape) for w in weights]
    out_specs = pl.BlockSpec((blk, 1), lambda i: (i, 0))
    return pl.pallas_call(
        _tc_dense_body,
        grid=grid,
        in_specs=in_specs,
        out_specs=out_specs,
        out_shape=jax.ShapeDtypeStruct((B, 1), jnp.float32),
    )(dense_t, ly_bm, *weights)


def kernel(dense_x, lS_i, lS_o, emb_tables,
           bot_W0, bot_b0, bot_W1, bot_b1, bot_W2, bot_b2,
           top_W0, top_b0, top_W1, top_b1, top_W2, top_b2):
    del lS_o
    offs = (jnp.arange(NTAB, dtype=jnp.int32) * (SEGP * D8))[:, None]
    idx4 = _permute_idx(lS_i) + offs
    idx4 = idx4.reshape(NTAB, NW, 2, NSUB, 128).reshape(
        NTAB * NW * 2, NSUB, 128)
    emb_t = jnp.transpose(emb_tables, (0, 2, 1))
    emb_flat = _tc_transpose(emb_t).reshape(NTAB * SEGP * D8, M // 2)
    ly_bm = _sc_embed(idx4, emb_flat)
    dense_t = dense_x.T
    weights = (bot_W0, bot_b0, bot_W1, bot_b1, bot_W2, bot_b2,
               top_W0, top_b0, top_W1, top_b1, top_W2, top_b2)
    return _tc_dense(dense_t, ly_bm, weights)


# speedup vs baseline: 165.3676x; 1.0132x over previous
import functools

import jax
import jax.numpy as jnp
import numpy as np
from jax import lax
from jax.experimental import pallas as pl
from jax.experimental.pallas import tpu as pltpu
from jax.experimental.pallas import tpu_sc as plsc

B = 4096
L = 20
NTAB = 26
VOCAB = 100000
M = 32
NC = 2
NS = 16
NW = NC * NS
BAGS_W = B // NW
CH = BAGS_W // 2
CH_ROWS = CH * L
NSUB = CH_ROWS // 128


def _sc_embed_body(idx_hbm, emb_hbm, out_hbm,
                   idx_v0, idx_v1, rows0, rows1, out_v, sem0, sem1,
                   g0, ng):
    w = lax.axis_index("s") * NC + lax.axis_index("c")

    def fire(chunk, idx_v, rows, sem):
        pltpu.sync_copy(idx_hbm.at[chunk], idx_v)
        for c in range(NSUB):
            pltpu.async_copy(emb_hbm.at[idx_v.at[c]],
                             rows.at[pl.ds(c * 128, 128)], sem)

    def drain(idx_v, rows, sem):
        for c in range(NSUB):
            pltpu.make_async_copy(emb_hbm.at[idx_v.at[0]],
                                  rows.at[pl.ds(c * 128, 128)], sem).wait()

    hi_mask = jnp.full((16,), -65536, jnp.int32)
    sh16 = jnp.full((16,), 16, jnp.int32)

    def split_row(vi):
        even = lax.bitcast_convert_type(lax.shift_left(vi, sh16), jnp.float32)
        odd = lax.bitcast_convert_type(jnp.bitwise_and(vi, hi_mask), jnp.float32)
        return even, odd

    def pool(rows, out_base):
        def bag_body(b, carry):
            base = b * L
            a0, a1 = split_row(rows[base, :])
            for r in range(1, L):
                b0, b1 = split_row(rows[base + r, :])
                a0 = a0 + b0
                a1 = a1 + b1
            out_v[out_base + b, pl.ds(0, 16)] = a0
            out_v[out_base + b, pl.ds(16, 16)] = a1
            return carry
        lax.fori_loop(0, CH, bag_body, 0, unroll=False)

    fire((g0 * NW + w) * 2, idx_v0, rows0, sem0)

    def table_body(t, carry):
        tw2 = ((g0 + t) * NW + w) * 2
        fire(tw2 + 1, idx_v1, rows1, sem1)
        drain(idx_v0, rows0, sem0)
        pool(rows0, 0)

        @pl.when(t < ng - 1)
        def _():
            fire(tw2 + 2 * NW, idx_v0, rows0, sem0)

        drain(idx_v1, rows1, sem1)
        pool(rows1, CH)
        pltpu.sync_copy(out_v, out_hbm.at[t, pl.ds(w * BAGS_W, BAGS_W), :])
        return carry

    lax.fori_loop(0, ng, table_body, 0, unroll=False)


def _sc_embed(idx4, emb_flat, g0, ng):
    mesh = plsc.VectorSubcoreMesh(core_axis_name="c", subcore_axis_name="s")
    kfn = pl.kernel(
        functools.partial(_sc_embed_body, g0=g0, ng=ng),
        out_type=jax.ShapeDtypeStruct((ng, B, M), jnp.float32),
        mesh=mesh,
        scratch_types=[
            pltpu.VMEM((NSUB, 128), jnp.int32),
            pltpu.VMEM((NSUB, 128), jnp.int32),
            pltpu.VMEM((CH_ROWS, M // 2), jnp.int32),
            pltpu.VMEM((CH_ROWS, M // 2), jnp.int32),
            pltpu.VMEM((BAGS_W, M), jnp.float32),
            pltpu.SemaphoreType.DMA,
            pltpu.SemaphoreType.DMA,
        ],
        compiler_params=pltpu.CompilerParams(use_tc_tiling_on_sc=False),
    )
    return kfn(idx4, emb_flat)


JB = VOCAB
D8 = 8
SEG = JB // D8
SEGP = SEG + 4


def _tc_transpose_body(src_ref, out_ref):
    x = src_ref[0]
    v = lax.bitcast_convert_type(x, jnp.int32)
    rnd = v + 32768
    lo = lax.shift_right_logical(rnd[:M // 2], 16)
    hi = jnp.bitwise_and(rnd[M // 2:], -65536)
    w = jnp.bitwise_or(lo, hi)
    ch = SEG // 4
    for c in range(4):
        xs = jnp.concatenate(
            [w[:, dq * SEG + c * ch:dq * SEG + (c + 1) * ch] for dq in range(D8)],
            axis=0)
        out_ref[0, c * ch:(c + 1) * ch, :] = xs.T


def _tc_transpose(emb_t, g0, ng):
    return pl.pallas_call(
        _tc_transpose_body,
        grid=(ng,),
        in_specs=[pl.BlockSpec((1, M, JB), lambda t: (g0 + t, 0, 0))],
        out_specs=pl.BlockSpec((1, SEGP, 128), lambda t: (t, 0, 0)),
        out_shape=jax.ShapeDtypeStruct((ng, SEGP, 128), jnp.int32),
        compiler_params=pltpu.CompilerParams(vmem_limit_bytes=100 * 1024 * 1024),
    )(emb_t)


def _permute_idx(idx):
    return (idx % SEG) * D8 + idx // SEG


def _tc_dense_body(dense_t, ly_bm,
                   w0, b0, w1, b1, w2, b2,
                   tw0, tb0, tw1, tb1, tw2, tb2, out_ref):
    d = dense_t[...]
    x = jax.nn.relu(jnp.dot(w0[...], d, preferred_element_type=jnp.float32)
                    + b0[...][:, None])
    x = jax.nn.relu(jnp.dot(w1[...], x, preferred_element_type=jnp.float32)
                    + b1[...][:, None])
    x = jax.nn.relu(jnp.dot(w2[...], x, preferred_element_type=jnp.float32)
                    + b2[...][:, None])
    ly3 = ly_bm[...]
    ly_t = jnp.transpose(ly3, (0, 2, 1)).reshape(NTAB * M, ly3.shape[1])
    r = jnp.concatenate([x, ly_t], axis=0)
    blk = r.shape[1]
    pieces = [x]
    for i in range(1, NTAB + 1):
        u = r[i * M:(i + 1) * M]
        r3 = r[:i * M].reshape(i, M, blk)
        pi = (r3 * u[None]).sum(axis=1)
        pieces.append(pi)
    z = jnp.concatenate(pieces, axis=0)
    h = jax.nn.relu(jnp.dot(tw0[...], z, preferred_element_type=jnp.float32)
                    + tb0[...][:, None])
    h = jax.nn.relu(jnp.dot(tw1[...], h, preferred_element_type=jnp.float32)
                    + tb1[...][:, None])
    p = jax.nn.sigmoid(jnp.dot(tw2[...], h, preferred_element_type=jnp.float32)
                       + tb2[...][:, None])
    out_ref[...] = p.T


def _tc_dense(dense_t, ly_bm, weights):
    blk = 512
    grid = (B // blk,)
    full = lambda shape: pl.BlockSpec(shape, lambda i: (0,) * len(shape))
    in_specs = [
        pl.BlockSpec((13, blk), lambda i: (0, i)),
        pl.BlockSpec((NTAB, blk, M), lambda i: (0, i, 0)),
    ] + [full(w.shape) for w in weights]
    out_specs = pl.BlockSpec((blk, 1), lambda i: (i, 0))
    return pl.pallas_call(
        _tc_dense_body,
        grid=grid,
        in_specs=in_specs,
        out_specs=out_specs,
        out_shape=jax.ShapeDtypeStruct((B, 1), jnp.float32),
    )(dense_t, ly_bm, *weights)


def kernel(dense_x, lS_i, lS_o, emb_tables,
           bot_W0, bot_b0, bot_W1, bot_b1, bot_W2, bot_b2,
           top_W0, top_b0, top_W1, top_b1, top_W2, top_b2):
    del lS_o
    NG = NTAB // 2
    offs = ((jnp.arange(NTAB, dtype=jnp.int32) % NG) * (SEGP * D8))[:, None]
    idx4 = _permute_idx(lS_i) + offs
    idx4 = idx4.reshape(NTAB, NW, 2, NSUB, 128).reshape(
        NTAB * NW * 2, NSUB, 128)
    emb_t = jnp.transpose(emb_tables, (0, 2, 1))
    ly_parts = []
    for g in range(2):
        ef = _tc_transpose(emb_t, g * NG, NG).reshape(NG * SEGP * D8, M // 2)
        ly_parts.append(_sc_embed(idx4, ef, g * NG, NG))
    ly_bm = jnp.concatenate(ly_parts, axis=0)
    dense_t = dense_x.T
    weights = (bot_W0, bot_b0, bot_W1, bot_b1, bot_W2, bot_b2,
               top_W0, top_b0, top_W1, top_b1, top_W2, top_b2)
    return _tc_dense(dense_t, ly_bm, weights)


# speedup vs baseline: 197.5316x; 1.1945x over previous
import functools

import jax
import jax.numpy as jnp
import numpy as np
from jax import lax
from jax.experimental import pallas as pl
from jax.experimental.pallas import tpu as pltpu
from jax.experimental.pallas import tpu_sc as plsc

B = 4096
L = 20
NTAB = 26
VOCAB = 100000
M = 32
NC = 2
NS = 16
NW = NC * NS
BAGS_W = B // NW
CH = BAGS_W // 2
CH_ROWS = CH * L
NSUB = CH_ROWS // 128


def _sc_embed_body(idx_hbm, emb_hbm, out_hbm,
                   idx_v0, idx_v1, rows0, rows1, out_v, sem0, sem1,
                   g0, ng):
    w = lax.axis_index("s") * NC + lax.axis_index("c")

    def fire(chunk, idx_v, rows, sem):
        pltpu.sync_copy(idx_hbm.at[chunk], idx_v)
        for c in range(NSUB):
            pltpu.async_copy(emb_hbm.at[idx_v.at[c]],
                             rows.at[pl.ds(c * 128, 128)], sem)

    def drain(idx_v, rows, sem):
        for c in range(NSUB):
            pltpu.make_async_copy(emb_hbm.at[idx_v.at[0]],
                                  rows.at[pl.ds(c * 128, 128)], sem).wait()

    hi_mask = jnp.full((16,), -65536, jnp.int32)
    sh16 = jnp.full((16,), 16, jnp.int32)

    def split_row(vi):
        even = lax.bitcast_convert_type(lax.shift_left(vi, sh16), jnp.float32)
        odd = lax.bitcast_convert_type(jnp.bitwise_and(vi, hi_mask), jnp.float32)
        return even, odd

    def pool(rows, out_base):
        def bag_body(b, carry):
            base = b * L
            e0, o0 = split_row(rows[base, :])
            e1, o1 = split_row(rows[base + 1, :])
            for r in range(2, L, 2):
                c0, c1 = split_row(rows[base + r, :])
                d0, d1 = split_row(rows[base + r + 1, :])
                e0 = e0 + c0
                o0 = o0 + c1
                e1 = e1 + d0
                o1 = o1 + d1
            out_v[out_base + b, pl.ds(0, 16)] = e0 + e1
            out_v[out_base + b, pl.ds(16, 16)] = o0 + o1
            return carry
        lax.fori_loop(0, CH, bag_body, 0, unroll=False)

    fire((g0 * NW + w) * 2, idx_v0, rows0, sem0)

    def table_body(t, carry):
        tw2 = ((g0 + t) * NW + w) * 2
        fire(tw2 + 1, idx_v1, rows1, sem1)
        drain(idx_v0, rows0, sem0)
        pool(rows0, 0)

        @pl.when(t < ng - 1)
        def _():
            fire(tw2 + 2 * NW, idx_v0, rows0, sem0)

        drain(idx_v1, rows1, sem1)
        pool(rows1, CH)
        pltpu.sync_copy(out_v,
                        out_hbm.at[t, pl.ds(w * BAGS_W, BAGS_W), pl.ds(0, M)])
        return carry

    lax.fori_loop(0, ng, table_body, 0, unroll=False)


def _sc_embed(idx4, emb_flat, g0, ng):
    mesh = plsc.VectorSubcoreMesh(core_axis_name="c", subcore_axis_name="s")
    kfn = pl.kernel(
        functools.partial(_sc_embed_body, g0=g0, ng=ng),
        out_type=jax.ShapeDtypeStruct((ng, B, 128), jnp.float32),
        mesh=mesh,
        scratch_types=[
            pltpu.VMEM((NSUB, 128), jnp.int32),
            pltpu.VMEM((NSUB, 128), jnp.int32),
            pltpu.VMEM((CH_ROWS, M // 2), jnp.int32),
            pltpu.VMEM((CH_ROWS, M // 2), jnp.int32),
            pltpu.VMEM((BAGS_W, M), jnp.float32),
            pltpu.SemaphoreType.DMA,
            pltpu.SemaphoreType.DMA,
        ],
        compiler_params=pltpu.CompilerParams(use_tc_tiling_on_sc=False),
    )
    return kfn(idx4, emb_flat)


JB = VOCAB
D8 = 8
SEG = JB // D8
SEGP = SEG + 4


def _tc_transpose_body(src_ref, out_ref):
    x = src_ref[0]
    v = lax.bitcast_convert_type(x, jnp.int32)
    rnd = v + 32768
    lo = lax.shift_right_logical(rnd[:M // 2], 16)
    hi = jnp.bitwise_and(rnd[M // 2:], -65536)
    w = jnp.bitwise_or(lo, hi)
    ch = SEG // 4
    for c in range(4):
        xs = jnp.concatenate(
            [w[:, dq * SEG + c * ch:dq * SEG + (c + 1) * ch] for dq in range(D8)],
            axis=0)
        out_ref[0, c * ch:(c + 1) * ch, :] = xs.T


def _tc_transpose(emb_t, g0, ng):
    return pl.pallas_call(
        _tc_transpose_body,
        grid=(ng,),
        in_specs=[pl.BlockSpec((1, M, JB), lambda t: (g0 + t, 0, 0))],
        out_specs=pl.BlockSpec((1, SEGP, 128), lambda t: (t, 0, 0)),
        out_shape=jax.ShapeDtypeStruct((ng, SEGP, 128), jnp.int32),
        compiler_params=pltpu.CompilerParams(vmem_limit_bytes=100 * 1024 * 1024),
    )(emb_t)


def _permute_idx(idx):
    return (idx % SEG) * D8 + idx // SEG


def _tc_dense_body(dense_t, ly_a, ly_b,
                   w0, b0, w1, b1, w2, b2,
                   tw0, tb0, tw1, tb1, tw2, tb2, out_ref):
    d = dense_t[...]
    x = jax.nn.relu(jnp.dot(w0[...], d, preferred_element_type=jnp.float32)
                    + b0[...][:, None])
    x = jax.nn.relu(jnp.dot(w1[...], x, preferred_element_type=jnp.float32)
                    + b1[...][:, None])
    x = jax.nn.relu(jnp.dot(w2[...], x, preferred_element_type=jnp.float32)
                    + b2[...][:, None])
    ly3 = jnp.concatenate([ly_a[...][:, :, :M], ly_b[...][:, :, :M]],
                          axis=0)
    ly_t = jnp.transpose(ly3, (0, 2, 1)).reshape(NTAB * M, ly3.shape[1])
    r = jnp.concatenate([x, ly_t], axis=0)
    blk = r.shape[1]
    pieces = [x]
    for i in range(1, NTAB + 1):
        u = r[i * M:(i + 1) * M]
        r3 = r[:i * M].reshape(i, M, blk)
        pi = (r3 * u[None]).sum(axis=1)
        pieces.append(pi)
    z = jnp.concatenate(pieces, axis=0)
    h = jax.nn.relu(jnp.dot(tw0[...], z, preferred_element_type=jnp.float32)
                    + tb0[...][:, None])
    h = jax.nn.relu(jnp.dot(tw1[...], h, preferred_element_type=jnp.float32)
                    + tb1[...][:, None])
    p = jax.nn.sigmoid(jnp.dot(tw2[...], h, preferred_element_type=jnp.float32)
                       + tb2[...][:, None])
    out_ref[...] = p.T


def _tc_dense(dense_t, ly_a, ly_b, weights):
    blk = 512
    grid = (B // blk,)
    full = lambda shape: pl.BlockSpec(shape, lambda i: (0,) * len(shape))
    in_specs = [
        pl.BlockSpec((13, blk), lambda i: (0, i)),
        pl.BlockSpec((NTAB // 2, blk, 128), lambda i: (0, i, 0)),
        pl.BlockSpec((NTAB // 2, blk, 128), lambda i: (0, i, 0)),
    ] + [full(w.shape) for w in weights]
    out_specs = pl.BlockSpec((blk, 1), lambda i: (i, 0))
    return pl.pallas_call(
        _tc_dense_body,
        grid=grid,
        in_specs=in_specs,
        out_specs=out_specs,
        out_shape=jax.ShapeDtypeStruct((B, 1), jnp.float32),
        compiler_params=pltpu.CompilerParams(vmem_limit_bytes=100 * 1024 * 1024),
    )(dense_t, ly_a, ly_b, *weights)


def kernel(dense_x, lS_i, lS_o, emb_tables,
           bot_W0, bot_b0, bot_W1, bot_b1, bot_W2, bot_b2,
           top_W0, top_b0, top_W1, top_b1, top_W2, top_b2):
    del lS_o
    NG = NTAB // 2
    offs = ((jnp.arange(NTAB, dtype=jnp.int32) % NG) * (SEGP * D8))[:, None]
    idx4 = _permute_idx(lS_i) + offs
    idx4 = idx4.reshape(NTAB, NW, 2, NSUB, 128).reshape(
        NTAB * NW * 2, NSUB, 128)
    emb_t = jnp.transpose(emb_tables, (0, 2, 1))
    ly_parts = []
    for g in range(2):
        ef = _tc_transpose(emb_t, g * NG, NG).reshape(NG * SEGP * D8, M // 2)
        ly_parts.append(_sc_embed(idx4, ef, g * NG, NG))
    dense_t = dense_x.T
    weights = (bot_W0, bot_b0, bot_W1, bot_b1, bot_W2, bot_b2,
               top_W0, top_b0, top_W1, top_b1, top_W2, top_b2)
    return _tc_dense(dense_t, ly_parts[0], ly_parts[1], weights)
